# baseline (speedup 1.0000x reference)
import functools

import jax
import jax.numpy as jnp
from jax import lax
from jax.experimental import pallas as pl
from jax.experimental.pallas import tpu as pltpu
from jax.experimental.pallas import tpu_sc as plsc

_NUM_CORES = 2
_NUM_SUBCORES = 16
_LANES = 16
_NW = _NUM_CORES * _NUM_SUBCORES


@functools.lru_cache(maxsize=None)
def _make_pool(B, L, H):
  CH = 40
  NCH = L // CH
  RPW = B // _NW
  HALF = RPW // 4
  EW = 2 * H
  NCOL = H // _LANES

  mesh = plsc.VectorSubcoreMesh(core_axis_name="c", subcore_axis_name="s")

  @functools.partial(
      pl.kernel,
      mesh=mesh,
      compiler_params=pltpu.CompilerParams(use_tc_tiling_on_sc=False),
      out_type=jax.ShapeDtypeStruct((B, H), jnp.float32),
      scratch_types=[
          pltpu.VMEM((HALF, L), jnp.int32),
          pltpu.VMEM((2, L, EW), jnp.float32),
          pltpu.VMEM((RPW, H), jnp.float32),
          pltpu.SemaphoreType.DMA,
          pltpu.SemaphoreType.DMA,
      ],
  )
  def pool(x_hbm, emb_hbm, out_hbm, idx_v, rows_v, out_v, sem0, sem1):
    wid = lax.axis_index("s") * _NUM_CORES + lax.axis_index("c")
    base = wid * RPW
    inv = jnp.float32(1.0 / L)
    UNROLL = 8

    def issue(r, buf, sem):
      r = jnp.minimum(r, HALF - 1)
      copies = []
      for j in range(NCH):
        copies.append(
            pltpu.async_copy(
                emb_hbm.at[idx_v.at[r, pl.ds(j * CH, CH)]],
                rows_v.at[buf, pl.ds(j * CH, CH)],
                sem,
            ))
      return copies

    def accum(buf, out_row):
      def acc_body(i, accs):
        res = accs
        for u in range(UNROLL):
          t = i * UNROLL + u
          res = tuple(
              res[c] + rows_v[buf, t, pl.ds(c * _LANES, _LANES)]
              for c in range(NCOL))
        return res

      zero = jnp.zeros((_LANES,), jnp.float32)
      accs = lax.fori_loop(0, L // UNROLL, acc_body, (zero,) * NCOL)
      for c in range(NCOL):
        out_v[out_row, pl.ds(c * _LANES, _LANES)] = accs[c] * inv

    for half in range(RPW // HALF):
      pltpu.sync_copy(x_hbm.at[pl.ds(base + half * HALF, HALF)], idx_v)

      issue(jnp.int32(0), 0, sem0)

      def wait_chunks(sem):
        for j in range(NCH):
          pltpu.make_async_copy(
              emb_hbm.at[idx_v.at[0, pl.ds(j * CH, CH)]],
              rows_v.at[0, pl.ds(j * CH, CH)],
              sem,
          ).wait()

      def loop_body(k, _):
        r = 2 * k
        wait_chunks(sem0)
        issue(r + 1, 1, sem1)
        accum(0, half * HALF + r)
        wait_chunks(sem1)
        issue(r + 2, 0, sem0)
        accum(1, half * HALF + r + 1)
        return 0

      lax.fori_loop(0, HALF // 2, loop_body, 0)
      wait_chunks(sem0)

    pltpu.sync_copy(out_v, out_hbm.at[pl.ds(base, RPW)])

  return pool


@functools.lru_cache(maxsize=None)
def _make_mlp(B, H, H2, N):
  BLK = 1024

  def body(p_ref, w1_ref, b1_ref, w2_ref, b2_ref, w3_ref, b3_ref, o_ref):
    h = p_ref[...]
    h = jnp.maximum(
        jnp.dot(h, w1_ref[...], preferred_element_type=jnp.float32)
        + b1_ref[...], 0.0)
    h = jnp.maximum(
        jnp.dot(h, w2_ref[...], preferred_element_type=jnp.float32)
        + b2_ref[...], 0.0)
    o_ref[...] = jnp.dot(
        h, w3_ref[...], preferred_element_type=jnp.float32) + b3_ref[...]

  return pl.pallas_call(
      body,
      grid=(B // BLK,),
      in_specs=[
          pl.BlockSpec((BLK, H), lambda i: (i, 0)),
          pl.BlockSpec((H, H2), lambda i: (0, 0)),
          pl.BlockSpec((1, H2), lambda i: (0, 0)),
          pl.BlockSpec((H2, H2), lambda i: (0, 0)),
          pl.BlockSpec((1, H2), lambda i: (0, 0)),
          pl.BlockSpec((H2, N), lambda i: (0, 0)),
          pl.BlockSpec((1, N), lambda i: (0, 0)),
      ],
      out_specs=pl.BlockSpec((BLK, N), lambda i: (i, 0)),
      out_shape=jax.ShapeDtypeStruct((B, N), jnp.float32),
  )


def kernel(x, emb, W1, b1, W2, b2, W3, b3):
  B, L = x.shape
  H = emb.shape[1]
  H2 = W1.shape[1]
  N = W3.shape[1]
  emb_pad = jnp.pad(emb, ((0, 0), (0, H)))
  pooled = _make_pool(B, L, H)(x.astype(jnp.int32), emb_pad)
  return _make_mlp(B, H, H2, N)(
      pooled, W1, b1.reshape(1, H2), W2, b2.reshape(1, H2), W3,
      b3.reshape(1, N))


# speedup vs baseline: 1.5061x; 1.5061x over previous
import functools

import jax
import jax.numpy as jnp
from jax import lax
from jax.experimental import pallas as pl
from jax.experimental.pallas import tpu as pltpu
from jax.experimental.pallas import tpu_sc as plsc

_NUM_CORES = 2
_NUM_SUBCORES = 16
_LANES = 16
_NW = _NUM_CORES * _NUM_SUBCORES


@functools.lru_cache(maxsize=None)
def _make_pool(B, L, H):
  CH = 40
  NCH = L // CH
  RPW = B // _NW
  HALF = RPW // 4
  NCOL = H // _LANES

  mesh = plsc.VectorSubcoreMesh(core_axis_name="c", subcore_axis_name="s")

  @functools.partial(
      pl.kernel,
      mesh=mesh,
      compiler_params=pltpu.CompilerParams(use_tc_tiling_on_sc=False),
      out_type=jax.ShapeDtypeStruct((B, H), jnp.float32),
      scratch_types=[
          pltpu.VMEM((HALF, L), jnp.int32),
          pltpu.VMEM((4, L, H), jnp.float32),
          pltpu.VMEM((RPW, H), jnp.float32),
          pltpu.SemaphoreType.DMA,
          pltpu.SemaphoreType.DMA,
          pltpu.SemaphoreType.DMA,
          pltpu.SemaphoreType.DMA,
      ],
  )
  def pool(x_hbm, emb_hbm, out_hbm, idx_v, rows_v, out_v, sem0, sem1, sem2, sem3):
    wid = lax.axis_index("s") * _NUM_CORES + lax.axis_index("c")
    base = wid * RPW
    inv = jnp.float32(1.0 / L)
    UNROLL = 8

    def issue(r, buf, sem):
      r = jnp.minimum(r, HALF - 1)
      copies = []
      for j in range(NCH):
        copies.append(
            pltpu.async_copy(
                emb_hbm.at[idx_v.at[r, pl.ds(j * CH, CH)]],
                rows_v.at[buf, pl.ds(j * CH, CH)],
                sem,
            ))
      return copies

    def accum(buf, out_row):
      def acc_body(i, accs):
        res = accs
        for u in range(UNROLL):
          t = i * UNROLL + u
          res = tuple(
              res[c] + rows_v[buf, t, pl.ds(c * _LANES, _LANES)]
              for c in range(NCOL))
        return res

      zero = jnp.zeros((_LANES,), jnp.float32)
      accs = lax.fori_loop(0, L // UNROLL, acc_body, (zero,) * NCOL)
      for c in range(NCOL):
        out_v[out_row, pl.ds(c * _LANES, _LANES)] = accs[c] * inv

    sems = (sem0, sem1, sem2, sem3)

    def wait_chunks(sem):
      for j in range(NCH):
        pltpu.make_async_copy(
            emb_hbm.at[idx_v.at[0, pl.ds(j * CH, CH)]],
            rows_v.at[0, pl.ds(j * CH, CH)],
            sem,
        ).wait()

    for half in range(RPW // HALF):
      pltpu.sync_copy(x_hbm.at[pl.ds(base + half * HALF, HALF)], idx_v)

      issue(jnp.int32(0), 0, sem0)
      issue(jnp.int32(1), 1, sem1)

      def loop_body(k, _):
        r = 4 * k
        for u in range(4):
          wait_chunks(sems[u])
          issue(r + u + 2, (u + 2) % 4, sems[(u + 2) % 4])
          accum(u, half * HALF + r + u)
        return 0

      lax.fori_loop(0, HALF // 4, loop_body, 0)
      wait_chunks(sem0)
      wait_chunks(sem1)

    pltpu.sync_copy(out_v, out_hbm.at[pl.ds(base, RPW)])

  return pool


@functools.lru_cache(maxsize=None)
def _make_mlp(B, H, H2, N):
  BLK = 1024

  def body(p_ref, w1_ref, b1_ref, w2_ref, b2_ref, w3_ref, b3_ref, o_ref):
    h = p_ref[...]
    h = jnp.maximum(
        jnp.dot(h, w1_ref[...], preferred_element_type=jnp.float32)
        + b1_ref[...], 0.0)
    h = jnp.maximum(
        jnp.dot(h, w2_ref[...], preferred_element_type=jnp.float32)
        + b2_ref[...], 0.0)
    o_ref[...] = jnp.dot(
        h, w3_ref[...], preferred_element_type=jnp.float32) + b3_ref[...]

  return pl.pallas_call(
      body,
      grid=(B // BLK,),
      in_specs=[
          pl.BlockSpec((BLK, H), lambda i: (i, 0)),
          pl.BlockSpec((H, H2), lambda i: (0, 0)),
          pl.BlockSpec((1, H2), lambda i: (0, 0)),
          pl.BlockSpec((H2, H2), lambda i: (0, 0)),
          pl.BlockSpec((1, H2), lambda i: (0, 0)),
          pl.BlockSpec((H2, N), lambda i: (0, 0)),
          pl.BlockSpec((1, N), lambda i: (0, 0)),
      ],
      out_specs=pl.BlockSpec((BLK, N), lambda i: (i, 0)),
      out_shape=jax.ShapeDtypeStruct((B, N), jnp.float32),
  )


def kernel(x, emb, W1, b1, W2, b2, W3, b3):
  B, L = x.shape
  H = emb.shape[1]
  H2 = W1.shape[1]
  N = W3.shape[1]
  pooled = _make_pool(B, L, H)(x.astype(jnp.int32), emb)
  return _make_mlp(B, H, H2, N)(
      pooled, W1, b1.reshape(1, H2), W2, b2.reshape(1, H2), W3,
      b3.reshape(1, N))
